# Initial kernel scaffold; baseline (speedup 1.0000x reference)
import jax
import jax.numpy as jnp
from jax.experimental import pallas as pl


def kernel(feats, adj, adj_orig, We1, We2, Wn1, bn1, Wn2, bn2):
    raise NotImplementedError("write your pallas kernel here")


# speedup vs baseline: 79.3644x; 79.3644x over previous
import functools

import jax
import jax.numpy as jnp
from jax.experimental import pallas as pl

_REMOVE_RATIO = 0.1
_ADD_RATIO = 0.1


def _mm_body(a_ref, b_ref, bias_ref, o_ref, *, relu):
    acc = jnp.dot(a_ref[...], b_ref[...], preferred_element_type=jnp.float32)
    acc = acc + bias_ref[...]
    if relu:
        acc = jnp.maximum(acc, 0.0)
    o_ref[...] = acc


def _mm(a, b, bias=None, relu=False):
    m, k = a.shape
    k2, n = b.shape
    assert k == k2
    if bias is None:
        bias = jnp.zeros((n,), jnp.float32)
    bias2 = bias.reshape(1, n)
    bm = min(256, m)
    bn = min(512, n)
    grid = (m // bm, n // bn)
    return pl.pallas_call(
        functools.partial(_mm_body, relu=relu),
        grid=grid,
        in_specs=[
            pl.BlockSpec((bm, k), lambda i, j: (i, 0)),
            pl.BlockSpec((k, bn), lambda i, j: (0, j)),
            pl.BlockSpec((1, bn), lambda i, j: (0, j)),
        ],
        out_specs=pl.BlockSpec((bm, bn), lambda i, j: (i, j)),
        out_shape=jax.ShapeDtypeStruct((m, n), jnp.float32),
    )(a, b, bias2)


def _stats_body(l_ref, adj_ref, mn_ref, mx_ref, se_ref, *, blk):
    i = pl.program_id(0)
    n = l_ref.shape[1]
    row = i * blk + jax.lax.broadcasted_iota(jnp.int32, (blk, n), 0)
    col = jax.lax.broadcasted_iota(jnp.int32, (blk, n), 1)
    off = row != col
    l = l_ref[...]
    mn = jnp.min(jnp.where(off, l, jnp.inf))
    mx = jnp.max(jnp.where(off, l, -jnp.inf))
    se = jnp.sum(adj_ref[...])
    first = i == 0
    pmn = jnp.where(first, jnp.inf, mn_ref[...])
    pmx = jnp.where(first, -jnp.inf, mx_ref[...])
    pse = jnp.where(first, 0.0, se_ref[...])
    mn_ref[...] = jnp.minimum(pmn, mn)
    mx_ref[...] = jnp.maximum(pmx, mx)
    se_ref[...] = pse + se


def _stats(logits, adj):
    n = logits.shape[0]
    blk = 256
    vec = jax.ShapeDtypeStruct((1, 128), jnp.float32)
    return pl.pallas_call(
        functools.partial(_stats_body, blk=blk),
        grid=(n // blk,),
        in_specs=[
            pl.BlockSpec((blk, n), lambda i: (i, 0)),
            pl.BlockSpec((blk, n), lambda i: (i, 0)),
        ],
        out_specs=[
            pl.BlockSpec((1, 128), lambda i: (0, 0)),
            pl.BlockSpec((1, 128), lambda i: (0, 0)),
            pl.BlockSpec((1, 128), lambda i: (0, 0)),
        ],
        out_shape=[vec, vec, vec],
    )(logits, adj)


def _count_body(l_ref, adj_ref, p_ref, cd_ref, ca_ref, *, blk):
    i = pl.program_id(0)
    n = l_ref.shape[1]
    row = i * blk + jax.lax.broadcasted_iota(jnp.int32, (blk, n), 0)
    col = jax.lax.broadcasted_iota(jnp.int32, (blk, n), 1)
    off = row != col
    p = p_ref[...]
    mn = p[0, 0]
    mx = p[0, 1]
    t_del = p[0, 2]
    t_add = p[0, 3]
    norm = (l_ref[...] - mn) / (mx - mn + 1e-8)
    exist = adj_ref[...] > 0.0
    c_del = jnp.sum(((norm <= t_del) & exist & off).astype(jnp.int32))
    c_add = jnp.sum(((norm <= t_add) & (~exist) & off).astype(jnp.int32))
    first = i == 0
    pd = jnp.where(first, 0, cd_ref[...])
    pa = jnp.where(first, 0, ca_ref[...])
    cd_ref[...] = pd + c_del
    ca_ref[...] = pa + c_add


def _count(logits, adj, params):
    n = logits.shape[0]
    blk = 256
    vec = jax.ShapeDtypeStruct((1, 128), jnp.int32)
    return pl.pallas_call(
        functools.partial(_count_body, blk=blk),
        grid=(n // blk,),
        in_specs=[
            pl.BlockSpec((blk, n), lambda i: (i, 0)),
            pl.BlockSpec((blk, n), lambda i: (i, 0)),
            pl.BlockSpec((1, 128), lambda i: (0, 0)),
        ],
        out_specs=[
            pl.BlockSpec((1, 128), lambda i: (0, 0)),
            pl.BlockSpec((1, 128), lambda i: (0, 0)),
        ],
        out_shape=[vec, vec],
    )(logits, adj, params)


def _apply_body(l_ref, adj_ref, p_ref, new_ref, deg_ref, *, blk):
    i = pl.program_id(0)
    n = l_ref.shape[1]
    row = i * blk + jax.lax.broadcasted_iota(jnp.int32, (blk, n), 0)
    col = jax.lax.broadcasted_iota(jnp.int32, (blk, n), 1)
    off = row != col
    p = p_ref[...]
    mn = p[0, 0]
    mx = p[0, 1]
    t_del = p[0, 2]
    t_add = p[0, 3]
    del_en = p[0, 4] > 0.0
    add_en = p[0, 5] > 0.0
    norm = (l_ref[...] - mn) / (mx - mn + 1e-8)
    a = adj_ref[...]
    exist = a > 0.0
    del_m = (norm < t_del) & exist & off & del_en
    add_m = (norm >= t_add) & (~exist) & off & add_en
    new = jnp.where(del_m, 0.0, a)
    new = jnp.where(add_m, 1.0, new)
    new_ref[...] = new
    deg_ref[...] = jnp.sum(new, axis=1, keepdims=True) + 1.0


def _apply(logits, adj, params):
    n = logits.shape[0]
    blk = 256
    return pl.pallas_call(
        functools.partial(_apply_body, blk=blk),
        grid=(n // blk,),
        in_specs=[
            pl.BlockSpec((blk, n), lambda i: (i, 0)),
            pl.BlockSpec((blk, n), lambda i: (i, 0)),
            pl.BlockSpec((1, 128), lambda i: (0, 0)),
        ],
        out_specs=[
            pl.BlockSpec((blk, n), lambda i: (i, 0)),
            pl.BlockSpec((blk, 1), lambda i: (i, 0)),
        ],
        out_shape=[
            jax.ShapeDtypeStruct((n, n), jnp.float32),
            jax.ShapeDtypeStruct((n, 1), jnp.float32),
        ],
    )(logits, adj, params)


def _norm_body(new_ref, degr_ref, degc_ref, o_ref, *, blk):
    i = pl.program_id(0)
    n = new_ref.shape[1]
    row = i * blk + jax.lax.broadcasted_iota(jnp.int32, (blk, n), 0)
    col = jax.lax.broadcasted_iota(jnp.int32, (blk, n), 1)
    eye = (row == col).astype(jnp.float32)
    degr = degr_ref[...]
    degc = degc_ref[...]
    dr = jnp.where(degr > 0.0, jax.lax.rsqrt(degr), 0.0)
    dc = jnp.where(degc > 0.0, jax.lax.rsqrt(degc), 0.0)
    o_ref[...] = (new_ref[...] + eye) * dr * dc


def _normalize_adj(adj_new, deg):
    n = adj_new.shape[0]
    blk = 256
    deg_t = deg.reshape(1, n)
    return pl.pallas_call(
        functools.partial(_norm_body, blk=blk),
        grid=(n // blk,),
        in_specs=[
            pl.BlockSpec((blk, n), lambda i: (i, 0)),
            pl.BlockSpec((blk, 1), lambda i: (i, 0)),
            pl.BlockSpec((1, n), lambda i: (0, 0)),
        ],
        out_specs=pl.BlockSpec((blk, n), lambda i: (i, 0)),
        out_shape=jax.ShapeDtypeStruct((n, n), jnp.float32),
    )(adj_new, deg, deg_t)


def kernel(feats, adj, adj_orig, We1, We2, Wn1, bn1, Wn2, bn2):
    n = adj.shape[0]
    f32 = jnp.float32

    x1 = _mm(feats, We1)
    h = _mm(adj, x1, relu=True)
    we2p = jnp.pad(We2, ((0, 0), (0, 128 - We2.shape[1])))
    x2 = _mm(h, we2p)
    z = _mm(adj, x2, relu=True)
    logits_p = _mm(z, z.T)
    adj_logits = logits_p

    mn_v, mx_v, se_v = _stats(logits_p, adj_orig)
    mn, mx, s_exist = mn_v[0, 0], mx_v[0, 0], se_v[0, 0]
    num_exist = (s_exist * 0.5).astype(jnp.int32)
    num_delete = jnp.floor(num_exist.astype(f32) * _REMOVE_RATIO).astype(jnp.int32)
    n_add = jnp.floor(num_exist.astype(f32) * _ADD_RATIO).astype(jnp.int32)
    m_cand = jnp.int32(n * (n - 1) // 2) - num_exist

    k_del2 = 2 * num_delete
    k_add2 = 2 * (m_cand - n_add + 1)
    hi0 = jnp.int32(0x3F800000)
    lo0 = jnp.int32(-1)

    def body(_, state):
        lo_d, hi_d, lo_a, hi_a = state
        mid_d = lo_d + (hi_d - lo_d) // 2
        mid_a = lo_a + (hi_a - lo_a) // 2
        t_d = jax.lax.bitcast_convert_type(mid_d, f32)
        t_a = jax.lax.bitcast_convert_type(mid_a, f32)
        params = jnp.zeros((1, 128), f32).at[0, :4].set(
            jnp.stack([mn, mx, t_d, t_a])
        )
        cd, ca = _count(logits_p, adj_orig, params)
        ok_d = cd[0, 0] >= k_del2
        ok_a = ca[0, 0] >= k_add2
        hi_d2 = jnp.where(ok_d, mid_d, hi_d)
        lo_d2 = jnp.where(ok_d, lo_d, mid_d)
        hi_a2 = jnp.where(ok_a, mid_a, hi_a)
        lo_a2 = jnp.where(ok_a, lo_a, mid_a)
        return lo_d2, hi_d2, lo_a2, hi_a2

    lo_d, hi_d, lo_a, hi_a = jax.lax.fori_loop(
        0, 32, body, (lo0, hi0, lo0, hi0)
    )
    thr_del = jax.lax.bitcast_convert_type(hi_d, f32)
    thr_add = jax.lax.bitcast_convert_type(hi_a, f32)

    params = jnp.zeros((1, 128), f32).at[0, :6].set(
        jnp.stack(
            [
                mn,
                mx,
                thr_del,
                thr_add,
                (num_delete >= 1).astype(f32),
                (n_add >= 1).astype(f32),
            ]
        )
    )
    adj_new, deg = _apply(logits_p, adj_orig, params)
    a_norm = _normalize_adj(adj_new, deg)

    x1n = _mm(feats, Wn1)
    h2 = _mm(a_norm, x1n, bias=bn1, relu=True)
    ncls = Wn2.shape[1]
    wn2p = jnp.pad(Wn2, ((0, 0), (0, 128 - ncls)))
    bn2p = jnp.pad(bn2, (0, 128 - ncls))
    x2n = _mm(h2, wn2p)
    outp = _mm(a_norm, x2n, bias=bn2p)
    output = outp[:, :ncls]
    return output, adj_logits, adj_new
